# baseline (speedup 1.0000x reference)
import jax
import jax.numpy as jnp
from jax import lax
from jax.experimental import pallas as pl
from jax.experimental.pallas import tpu as pltpu
from jax.experimental.pallas import tpu_sc as plsc

CLASSES = 10199
BATCH = 16384
NC = 2
NS = 16
LANES = 16
NW = NC * NS
COLS_W = BATCH // NW
CHUNKS = COLS_W // LANES
KROWS = 120
NBLK = 84
TAILK = 112
TC0 = NBLK * KROWS + TAILK
BR = 2048


def _sc_body(cap_hbm, out_hbm, idx_v, buf0, buf1, sem0, sem1):
    wid = lax.axis_index("s") * NC + lax.axis_index("c")
    col0 = wid * COLS_W

    pltpu.sync_copy(cap_hbm.at[pl.ds(col0, COLS_W)], idx_v)

    zeros16 = jnp.zeros((LANES,), jnp.float32)
    ones16 = jnp.full((LANES,), 1.0, jnp.float32)
    iota16 = lax.iota(jnp.int32, LANES)

    def _zero(i, _):
        for r in range(KROWS):
            buf0[r, pl.ds(i * LANES, LANES)] = zeros16
            buf1[r, pl.ds(i * LANES, LANES)] = zeros16
        return 0

    lax.fori_loop(0, COLS_W // LANES, _zero, 0)

    def _sweep(buf, b_set, b_clear, set_hi):
        def _chunk(ch, _):
            caps = idx_v[pl.ds(ch * LANES, LANES)]
            cols = ch * LANES + iota16
            rel_s = caps - b_set * KROWS
            plsc.store_scatter(
                buf, [jnp.clip(rel_s, 0, KROWS - 1), cols], ones16,
                mask=(rel_s >= 0) & (rel_s < set_hi),
            )
            rel_c = caps - b_clear * KROWS
            plsc.store_scatter(
                buf, [jnp.clip(rel_c, 0, KROWS - 1), cols], zeros16,
                mask=(rel_c >= 0) & (rel_c < KROWS),
            )
            return 0

        lax.fori_loop(0, CHUNKS, _chunk, 0)

    def _start(buf, sem, b, k):
        pltpu.make_async_copy(
            buf.at[pl.ds(0, k), :],
            out_hbm.at[pl.ds(b * KROWS, k), pl.ds(col0, COLS_W)],
            sem,
        ).start()

    def _wait(buf, sem, b, k):
        pltpu.make_async_copy(
            buf.at[pl.ds(0, k), :],
            out_hbm.at[pl.ds(b * KROWS, k), pl.ds(col0, COLS_W)],
            sem,
        ).wait()

    _sweep(buf0, jnp.int32(0), jnp.int32(-2), KROWS)
    _start(buf0, sem0, 0, KROWS)
    _sweep(buf1, jnp.int32(1), jnp.int32(-1), KROWS)
    _start(buf1, sem1, 1, KROWS)

    def _iter(p, _):
        b = 2 * p
        _wait(buf0, sem0, b - 2, KROWS)
        _sweep(buf0, b, b - 2, KROWS)
        _start(buf0, sem0, b, KROWS)
        _wait(buf1, sem1, b - 1, KROWS)
        _sweep(buf1, b + 1, b - 1, KROWS)
        _start(buf1, sem1, b + 1, KROWS)
        return 0

    lax.fori_loop(1, NBLK // 2, _iter, 0)

    _wait(buf0, sem0, NBLK - 2, KROWS)
    _sweep(buf0, jnp.int32(NBLK), jnp.int32(NBLK - 2), TAILK)
    _start(buf0, sem0, NBLK, TAILK)

    _wait(buf1, sem1, NBLK - 1, KROWS)
    _wait(buf0, sem0, NBLK, TAILK)


def _tc_tail(cap_ref, alias_ref, out_ref):
    del alias_ref
    cap = cap_ref[...]
    classes = TC0 + jax.lax.broadcasted_iota(jnp.int32, (8, BR), 0)
    out_ref[...] = (classes == cap[None, :]).astype(jnp.float32)


def kernel(obj, caption, puzzle):
    cap = caption[0]
    mesh = plsc.VectorSubcoreMesh(
        core_axis_name="c", subcore_axis_name="s", num_cores=NC, num_subcores=NS
    )
    out_t = pl.kernel(
        _sc_body,
        out_type=jax.ShapeDtypeStruct((CLASSES, BATCH), jnp.float32),
        mesh=mesh,
        compiler_params=pltpu.CompilerParams(
            needs_layout_passes=False,
            use_tc_tiling_on_sc=True,
        ),
        scratch_types=[
            pltpu.VMEM((COLS_W,), jnp.int32),
            pltpu.VMEM((KROWS, COLS_W), jnp.float32),
            pltpu.VMEM((KROWS, COLS_W), jnp.float32),
            pltpu.SemaphoreType.DMA,
            pltpu.SemaphoreType.DMA,
        ],
    )(cap)

    out_t = pl.pallas_call(
        _tc_tail,
        grid=(BATCH // BR,),
        in_specs=[
            pl.BlockSpec((BR,), lambda j: (j,)),
            pl.BlockSpec(memory_space=pl.ANY),
        ],
        out_specs=pl.BlockSpec((8, BR), lambda j: (TC0 // 8, j)),
        out_shape=jax.ShapeDtypeStruct((CLASSES, BATCH), jnp.float32),
        input_output_aliases={1: 0},
    )(cap, out_t)
    return out_t.T


# speedup vs baseline: 1.0047x; 1.0047x over previous
import jax
import jax.numpy as jnp
from jax import lax
from jax.experimental import pallas as pl
from jax.experimental.pallas import tpu as pltpu
from jax.experimental.pallas import tpu_sc as plsc

CLASSES = 10199
BATCH = 16384
NC = 2
NS = 16
LANES = 16
NW = NC * NS
COLS_W = BATCH // NW
CHUNKS = COLS_W // LANES
KROWS = 64
NBLK = 159
TAILK = 16
TC0 = NBLK * KROWS + TAILK
BR = 2048


def _sc_body(cap_hbm, out_hbm, idx_v, buf0, buf1, buf2, sem0, sem1, sem2):
    wid = lax.axis_index("s") * NC + lax.axis_index("c")
    col0 = wid * COLS_W
    bufs = (buf0, buf1, buf2)
    sems = (sem0, sem1, sem2)

    pltpu.sync_copy(cap_hbm.at[pl.ds(col0, COLS_W)], idx_v)

    zeros16 = jnp.zeros((LANES,), jnp.float32)
    ones16 = jnp.full((LANES,), 1.0, jnp.float32)
    iota16 = lax.iota(jnp.int32, LANES)

    def _zero(i, _):
        for r in range(KROWS):
            for buf in bufs:
                buf[r, pl.ds(i * LANES, LANES)] = zeros16
        return 0

    lax.fori_loop(0, COLS_W // LANES, _zero, 0)

    def _sweep(buf, b_set, b_clear, set_hi):
        def _chunk(ch, _):
            caps = idx_v[pl.ds(ch * LANES, LANES)]
            cols = ch * LANES + iota16
            rel_s = caps - b_set * KROWS
            plsc.store_scatter(
                buf, [jnp.clip(rel_s, 0, KROWS - 1), cols], ones16,
                mask=(rel_s >= 0) & (rel_s < set_hi),
            )
            rel_c = caps - b_clear * KROWS
            plsc.store_scatter(
                buf, [jnp.clip(rel_c, 0, KROWS - 1), cols], zeros16,
                mask=(rel_c >= 0) & (rel_c < KROWS),
            )
            return 0

        lax.fori_loop(0, CHUNKS, _chunk, 0)

    def _start(buf, sem, b, k):
        pltpu.make_async_copy(
            buf.at[pl.ds(0, k), :],
            out_hbm.at[pl.ds(b * KROWS, k), pl.ds(col0, COLS_W)],
            sem,
        ).start()

    def _wait(buf, sem, b, k):
        pltpu.make_async_copy(
            buf.at[pl.ds(0, k), :],
            out_hbm.at[pl.ds(b * KROWS, k), pl.ds(col0, COLS_W)],
            sem,
        ).wait()

    for j in range(3):
        _sweep(bufs[j], jnp.int32(j), jnp.int32(j - 3), KROWS)
        _start(bufs[j], sems[j], j, KROWS)

    def _iter(p, _):
        for j in range(3):
            b = 3 * p + j
            _wait(bufs[j], sems[j], b - 3, KROWS)
            _sweep(bufs[j], b, b - 3, KROWS)
            _start(bufs[j], sems[j], b, KROWS)
        return 0

    lax.fori_loop(1, NBLK // 3, _iter, 0)

    _wait(buf0, sem0, NBLK - 3, KROWS)
    _sweep(buf0, jnp.int32(NBLK), jnp.int32(NBLK - 3), TAILK)
    _start(buf0, sem0, NBLK, TAILK)

    _wait(buf1, sem1, NBLK - 2, KROWS)
    _wait(buf2, sem2, NBLK - 1, KROWS)
    _wait(buf0, sem0, NBLK, TAILK)


def _tc_tail(cap_ref, alias_ref, out_ref):
    del alias_ref
    cap = cap_ref[...]
    classes = TC0 + jax.lax.broadcasted_iota(jnp.int32, (8, BR), 0)
    out_ref[...] = (classes == cap[None, :]).astype(jnp.float32)


def kernel(obj, caption, puzzle):
    cap = caption[0]
    mesh = plsc.VectorSubcoreMesh(
        core_axis_name="c", subcore_axis_name="s", num_cores=NC, num_subcores=NS
    )
    out_t = pl.kernel(
        _sc_body,
        out_type=jax.ShapeDtypeStruct((CLASSES, BATCH), jnp.float32),
        mesh=mesh,
        compiler_params=pltpu.CompilerParams(
            needs_layout_passes=False,
            use_tc_tiling_on_sc=True,
        ),
        scratch_types=[
            pltpu.VMEM((COLS_W,), jnp.int32),
            pltpu.VMEM((KROWS, COLS_W), jnp.float32),
            pltpu.VMEM((KROWS, COLS_W), jnp.float32),
            pltpu.VMEM((KROWS, COLS_W), jnp.float32),
            pltpu.SemaphoreType.DMA,
            pltpu.SemaphoreType.DMA,
            pltpu.SemaphoreType.DMA,
        ],
    )(cap)

    out_t = pl.pallas_call(
        _tc_tail,
        grid=(BATCH // BR,),
        in_specs=[
            pl.BlockSpec((BR,), lambda j: (j,)),
            pl.BlockSpec(memory_space=pl.ANY),
        ],
        out_specs=pl.BlockSpec((8, BR), lambda j: (TC0 // 8, j)),
        out_shape=jax.ShapeDtypeStruct((CLASSES, BATCH), jnp.float32),
        input_output_aliases={1: 0},
    )(cap, out_t)
    return out_t.T


# speedup vs baseline: 1.0218x; 1.0171x over previous
import jax
import jax.numpy as jnp
from jax import lax
from jax.experimental import pallas as pl
from jax.experimental.pallas import tpu as pltpu
from jax.experimental.pallas import tpu_sc as plsc

CLASSES = 10199
BATCH = 16384
NC = 2
NS = 16
LANES = 16
NW = NC * NS
COLS_W = BATCH // NW
CHUNKS = COLS_W // LANES
KROWS = 64
NBLK = 159
TAILK = 16
TC0 = NBLK * KROWS + TAILK
BR = 8192


def _sc_body(cap_hbm, out_hbm, idx_v, buf0, buf1, sem0, sem1):
    wid = lax.axis_index("s") * NC + lax.axis_index("c")
    col0 = wid * COLS_W

    pltpu.sync_copy(cap_hbm.at[pl.ds(col0, COLS_W)], idx_v)

    zeros16 = jnp.zeros((LANES,), jnp.float32)
    ones16 = jnp.full((LANES,), 1.0, jnp.float32)
    iota16 = lax.iota(jnp.int32, LANES)

    def _zero(i, _):
        for r in range(KROWS):
            buf0[r, pl.ds(i * LANES, LANES)] = zeros16
            buf1[r, pl.ds(i * LANES, LANES)] = zeros16
        return 0

    lax.fori_loop(0, COLS_W // LANES, _zero, 0)

    def _sweep(buf, b_set, b_clear, set_hi):
        def _chunk(ch, _):
            caps = idx_v[pl.ds(ch * LANES, LANES)]
            cols = ch * LANES + iota16
            rel_s = caps - b_set * KROWS
            plsc.store_scatter(
                buf, [jnp.clip(rel_s, 0, KROWS - 1), cols], ones16,
                mask=(rel_s >= 0) & (rel_s < set_hi),
            )
            rel_c = caps - b_clear * KROWS
            plsc.store_scatter(
                buf, [jnp.clip(rel_c, 0, KROWS - 1), cols], zeros16,
                mask=(rel_c >= 0) & (rel_c < KROWS),
            )
            return 0

        lax.fori_loop(0, CHUNKS, _chunk, 0)

    def _start(buf, sem, b, k):
        pltpu.make_async_copy(
            buf.at[pl.ds(0, k), :],
            out_hbm.at[pl.ds(b * KROWS, k), pl.ds(col0, COLS_W)],
            sem,
        ).start()

    def _wait(buf, sem, b, k):
        pltpu.make_async_copy(
            buf.at[pl.ds(0, k), :],
            out_hbm.at[pl.ds(b * KROWS, k), pl.ds(col0, COLS_W)],
            sem,
        ).wait()

    _sweep(buf0, jnp.int32(0), jnp.int32(-2), KROWS)
    _start(buf0, sem0, 0, KROWS)
    _sweep(buf1, jnp.int32(1), jnp.int32(-1), KROWS)
    _start(buf1, sem1, 1, KROWS)

    def _iter(p, _):
        b = 2 * p
        _wait(buf0, sem0, b - 2, KROWS)
        _sweep(buf0, b, b - 2, KROWS)
        _start(buf0, sem0, b, KROWS)
        _wait(buf1, sem1, b - 1, KROWS)
        _sweep(buf1, b + 1, b - 1, KROWS)
        _start(buf1, sem1, b + 1, KROWS)
        return 0

    lax.fori_loop(1, NBLK // 2, _iter, 0)

    _wait(buf0, sem0, NBLK - 3, KROWS)
    _sweep(buf0, jnp.int32(NBLK - 1), jnp.int32(NBLK - 3), KROWS)
    _start(buf0, sem0, NBLK - 1, KROWS)
    _wait(buf1, sem1, NBLK - 2, KROWS)
    _sweep(buf1, jnp.int32(NBLK), jnp.int32(NBLK - 2), TAILK)
    _start(buf1, sem1, NBLK, TAILK)

    _wait(buf0, sem0, NBLK - 1, KROWS)
    _wait(buf1, sem1, NBLK, TAILK)


def _tc_tail(cap_ref, alias_ref, out_ref):
    del alias_ref
    cap = cap_ref[...]
    classes = TC0 + jax.lax.broadcasted_iota(jnp.int32, (8, BR), 0)
    out_ref[...] = (classes == cap[None, :]).astype(jnp.float32)


def kernel(obj, caption, puzzle):
    cap = caption[0]
    mesh = plsc.VectorSubcoreMesh(
        core_axis_name="c", subcore_axis_name="s", num_cores=NC, num_subcores=NS
    )
    out_t = pl.kernel(
        _sc_body,
        out_type=jax.ShapeDtypeStruct((CLASSES, BATCH), jnp.float32),
        mesh=mesh,
        compiler_params=pltpu.CompilerParams(
            needs_layout_passes=False,
            use_tc_tiling_on_sc=True,
        ),
        scratch_types=[
            pltpu.VMEM((COLS_W,), jnp.int32),
            pltpu.VMEM((KROWS, COLS_W), jnp.float32),
            pltpu.VMEM((KROWS, COLS_W), jnp.float32),
            pltpu.SemaphoreType.DMA,
            pltpu.SemaphoreType.DMA,
        ],
    )(cap)

    out_t = pl.pallas_call(
        _tc_tail,
        grid=(BATCH // BR,),
        in_specs=[
            pl.BlockSpec((BR,), lambda j: (j,)),
            pl.BlockSpec(memory_space=pl.ANY),
        ],
        out_specs=pl.BlockSpec((8, BR), lambda j: (TC0 // 8, j)),
        out_shape=jax.ShapeDtypeStruct((CLASSES, BATCH), jnp.float32),
        input_output_aliases={1: 0},
    )(cap, out_t)
    return out_t.T
